# baseline (speedup 1.0000x reference)
import functools

import jax
import jax.numpy as jnp
from jax import lax
from jax.experimental import pallas as pl
from jax.experimental.pallas import tpu as pltpu
from jax.experimental.pallas import tpu_sc as plsc

S = 7
B = 2
C = 20
LEN = 5 * B + C
BS = 64
N_CELLS = BS * S * S
L = 16
NS = 16
CPT = N_CELLS // NS
BPT = BS // NS
FULL = CPT // L
TAIL = CPT - FULL * L

_f32 = jnp.float32


def _accum_losses(pvm, tvm, off, wt, accvm):
    lb, rr, cc = off

    def pcol(c):
        return plsc.load_gather(
            pvm, [lb, rr, cc, jnp.full((L,), c, jnp.int32)])

    def tcol(c):
        return plsc.load_gather(
            tvm, [lb, rr, cc, jnp.full((L,), c, jnp.int32)])

    tc4 = tcol(4)
    tc9 = tcol(9)
    pc0 = pcol(4)
    pc1 = pcol(9)

    noo_f = jnp.where(tc4 == _f32(0.0), _f32(1.0), _f32(0.0))
    d0 = pc0 - tc4
    d1 = pc1 - tc9
    noo = _f32(0.5) * noo_f * (d0 * d0 + d1 * d1)
    if wt is not None:
        noo = jnp.where(wt, noo, _f32(0.0))
    accvm[...] = accvm[...] + noo

    @pl.when(jnp.max(tc4) > _f32(0.0))
    def _():
        coo = tc4 > _f32(0.0)
        coo_f = jnp.where(coo, _f32(1.0), _f32(0.0))

        tx, ty, tw, th = tcol(0), tcol(1), tcol(2), tcol(3)
        t1x = tx / _f32(S) - _f32(0.5) * tw
        t2x = tx / _f32(S) + _f32(0.5) * tw
        t1y = ty / _f32(S) - _f32(0.5) * th
        t2y = ty / _f32(S) + _f32(0.5) * th
        a2 = (t2x - t1x) * (t2y - t1y)

        def iou(px, py, pw, ph):
            p1x = px / _f32(S) - _f32(0.5) * pw
            p2x = px / _f32(S) + _f32(0.5) * pw
            p1y = py / _f32(S) - _f32(0.5) * ph
            p2y = py / _f32(S) + _f32(0.5) * ph
            wx = jnp.maximum(
                jnp.minimum(p2x, t2x) - jnp.maximum(p1x, t1x), _f32(0.0))
            wy = jnp.maximum(
                jnp.minimum(p2y, t2y) - jnp.maximum(p1y, t1y), _f32(0.0))
            inter = wx * wy
            a1 = (p2x - p1x) * (p2y - p1y)
            denom = a1 + a2 - inter
            safe = jnp.where(coo, denom, _f32(1.0))
            return inter / safe

        iou0 = iou(pcol(0), pcol(1), pcol(2), pcol(3))
        iou1 = iou(pcol(5), pcol(6), pcol(7), pcol(8))
        max_iou = jnp.maximum(iou0, iou1)
        resp_c = jnp.where(iou1 > iou0, pc1, pc0)
        dc = resp_c - max_iou
        contain = dc * dc

        cls = jnp.zeros((L,), _f32)
        for c in range(C):
            d = pcol(10 + c) - tcol(10 + c)
            cls = cls + d * d

        obj = coo_f * (contain + cls)
        if wt is not None:
            obj = jnp.where(wt, obj, _f32(0.0))
        accvm[...] = accvm[...] + obj


def _sc_body(pred_hbm, tgt_hbm, out_hbm, pvm, tvm, accvm, redvm, shared,
             sem_p, sem_t):
    sid = lax.axis_index("s")
    base = sid * BPT
    cp = pltpu.async_copy(pred_hbm.at[pl.ds(base, BPT)], pvm, sem_p)
    ct = pltpu.async_copy(tgt_hbm.at[pl.ds(base, BPT)], tvm, sem_t)
    cp.wait()
    ct.wait()

    accvm[...] = jnp.zeros((L,), _f32)
    lane = lax.iota(jnp.int32, L)

    def _split(cell):
        lb = cell // (S * S)
        rem = cell - lb * (S * S)
        rr = rem // S
        cc = rem - rr * S
        return lb, rr, cc

    def vec_body(k, carry):
        _accum_losses(pvm, tvm, _split(lane + k * L), None, accvm)
        return carry

    lax.fori_loop(0, FULL, vec_body, 0)
    tail_cell = jnp.minimum(lane, TAIL - 1) + FULL * L
    _accum_losses(pvm, tvm, _split(tail_cell), lane < TAIL, accvm)

    pltpu.sync_copy(accvm, shared.at[sid])
    plsc.subcore_barrier()

    @pl.when(sid == 0)
    def _():
        pltpu.sync_copy(shared, redvm)
        t = jnp.zeros((L,), _f32)
        for i in range(NS):
            t = t + redvm[i, :]
        total = jnp.sum(t) * _f32(1.0 / BS)
        accvm[...] = jnp.full((L,), total, _f32)
        pltpu.sync_copy(accvm, out_hbm)


_mesh = plsc.VectorSubcoreMesh(
    core_axis_name="c", subcore_axis_name="s", num_cores=1)

_sc_yolo = functools.partial(
    pl.kernel,
    out_type=jax.ShapeDtypeStruct((L,), _f32),
    mesh=_mesh,
    compiler_params=pltpu.CompilerParams(
        needs_layout_passes=False, use_tc_tiling_on_sc=False),
    scratch_types=[
        pltpu.VMEM((BPT, S, S, LEN), _f32),
        pltpu.VMEM((BPT, S, S, LEN), _f32),
        pltpu.VMEM((L,), _f32),
        pltpu.VMEM((NS, L), _f32),
        pltpu.VMEM_SHARED((NS, L), _f32),
        pltpu.SemaphoreType.DMA,
        pltpu.SemaphoreType.DMA,
    ],
)(_sc_body)


def kernel(prediction, target):
    out = _sc_yolo(prediction, target)
    return out[0]


# speedup vs baseline: 1.0757x; 1.0757x over previous
import functools

import jax
import jax.numpy as jnp
from jax import lax
from jax.experimental import pallas as pl
from jax.experimental.pallas import tpu as pltpu
from jax.experimental.pallas import tpu_sc as plsc

S = 7
B = 2
C = 20
LEN = 5 * B + C
BS = 64
L = 16
NS = 16
NSLAB = S * S
NG = BS // L
NU = NSLAB * NG
NBUF = 4

_f32 = jnp.float32


def _unit_losses(pvm, tvm, b0, accvm):
    def pcol(c):
        return pvm[c, pl.ds(b0, L)]

    def tcol(c):
        return tvm[c, pl.ds(b0, L)]

    tc4 = tcol(4)
    tc9 = tcol(9)
    pc0 = pcol(4)
    pc1 = pcol(9)

    noo_f = jnp.where(tc4 == _f32(0.0), _f32(1.0), _f32(0.0))
    d0 = pc0 - tc4
    d1 = pc1 - tc9
    accvm[...] = accvm[...] + _f32(0.5) * noo_f * (d0 * d0 + d1 * d1)

    @pl.when(jnp.max(tc4) > _f32(0.0))
    def _():
        coo = tc4 > _f32(0.0)
        coo_f = jnp.where(coo, _f32(1.0), _f32(0.0))

        tx, ty, tw, th = tcol(0), tcol(1), tcol(2), tcol(3)
        t1x = tx / _f32(S) - _f32(0.5) * tw
        t2x = tx / _f32(S) + _f32(0.5) * tw
        t1y = ty / _f32(S) - _f32(0.5) * th
        t2y = ty / _f32(S) + _f32(0.5) * th
        a2 = (t2x - t1x) * (t2y - t1y)

        def iou(px, py, pw, ph):
            p1x = px / _f32(S) - _f32(0.5) * pw
            p2x = px / _f32(S) + _f32(0.5) * pw
            p1y = py / _f32(S) - _f32(0.5) * ph
            p2y = py / _f32(S) + _f32(0.5) * ph
            wx = jnp.maximum(
                jnp.minimum(p2x, t2x) - jnp.maximum(p1x, t1x), _f32(0.0))
            wy = jnp.maximum(
                jnp.minimum(p2y, t2y) - jnp.maximum(p1y, t1y), _f32(0.0))
            inter = wx * wy
            a1 = (p2x - p1x) * (p2y - p1y)
            denom = a1 + a2 - inter
            safe = jnp.where(coo, denom, _f32(1.0))
            return inter / safe

        iou0 = iou(pcol(0), pcol(1), pcol(2), pcol(3))
        iou1 = iou(pcol(5), pcol(6), pcol(7), pcol(8))
        max_iou = jnp.maximum(iou0, iou1)
        resp_c = jnp.where(iou1 > iou0, pc1, pc0)
        dc = resp_c - max_iou
        contain = dc * dc

        cls = jnp.zeros((L,), _f32)
        for c in range(C):
            d = pcol(10 + c) - tcol(10 + c)
            cls = cls + d * d

        accvm[...] = accvm[...] + coo_f * (contain + cls)


def _sc_body(pred_hbm, tgt_hbm, out_hbm, pvms, tvms, accvm, redvm, shared,
             sem_p, sem_t):
    sid = lax.axis_index("s")
    slab0 = 3 * sid
    last = sid == NS - 1

    copies = []
    for j in range(NBUF):
        s = slab0 + j if j < 3 else jnp.minimum(slab0 + j, NSLAB - 1)
        copies.append(pltpu.async_copy(pred_hbm.at[s], pvms[j], sem_p))
        copies.append(pltpu.async_copy(tgt_hbm.at[s], tvms[j], sem_t))
    for cp in copies:
        cp.wait()

    accvm[...] = jnp.zeros((L,), _f32)

    for j in range(3):
        for g in range(NG):
            _unit_losses(pvms[j], tvms[j], g * L, accvm)

    @pl.when(last)
    def _():
        for g in range(NG):
            _unit_losses(pvms[3], tvms[3], g * L, accvm)

    pltpu.sync_copy(accvm, shared.at[pl.ds(sid * L, L)])
    plsc.subcore_barrier()

    @pl.when(sid == 0)
    def _():
        pltpu.sync_copy(shared, redvm)
        t = jnp.zeros((L,), _f32)
        for i in range(NS):
            t = t + redvm[pl.ds(i * L, L)]
        total = jnp.sum(t) * _f32(1.0 / BS)
        accvm[...] = jnp.full((L,), total, _f32)
        pltpu.sync_copy(accvm, out_hbm)


_mesh = plsc.VectorSubcoreMesh(
    core_axis_name="c", subcore_axis_name="s", num_cores=1)

_sc_yolo = functools.partial(
    pl.kernel,
    out_type=jax.ShapeDtypeStruct((L,), _f32),
    mesh=_mesh,
    compiler_params=pltpu.CompilerParams(
        needs_layout_passes=False, use_tc_tiling_on_sc=True),
    scratch_types=[
        [pltpu.VMEM((LEN, BS), _f32)] * NBUF,
        [pltpu.VMEM((LEN, BS), _f32)] * NBUF,
        pltpu.VMEM((L,), _f32),
        pltpu.VMEM((NS * L,), _f32),
        pltpu.VMEM_SHARED((NS * L,), _f32),
        pltpu.SemaphoreType.DMA,
        pltpu.SemaphoreType.DMA,
    ],
)(_sc_body)


def kernel(prediction, target):
    qp = jnp.transpose(prediction, (1, 2, 3, 0)).reshape(NSLAB, LEN, BS)
    qt = jnp.transpose(target, (1, 2, 3, 0)).reshape(NSLAB, LEN, BS)
    out = _sc_yolo(qp, qt)
    return out[0]


# speedup vs baseline: 1.2145x; 1.1290x over previous
import functools

import jax
import jax.numpy as jnp
from jax import lax
from jax.experimental import pallas as pl
from jax.experimental.pallas import tpu as pltpu
from jax.experimental.pallas import tpu_sc as plsc

S = 7
B = 2
C = 20
LEN = 5 * B + C
BS = 64
L = 16
NS = 16
NSLAB = S * S
NG = BS // L
NU = NSLAB * NG
NBUF = 5

_f32 = jnp.float32


def _unit_losses(pvm, tvm, jv, bv, accvm):
    def pcol(c):
        return plsc.load_gather(pvm, [jv + c, bv])

    def tcol(c):
        return plsc.load_gather(tvm, [jv + c, bv])

    tc4 = tcol(4)
    tc9 = tcol(9)
    pc0 = pcol(4)
    pc1 = pcol(9)

    noo_f = jnp.where(tc4 == _f32(0.0), _f32(1.0), _f32(0.0))
    d0 = pc0 - tc4
    d1 = pc1 - tc9
    accvm[...] = accvm[...] + _f32(0.5) * noo_f * (d0 * d0 + d1 * d1)

    @pl.when(jnp.max(tc4) > _f32(0.0))
    def _():
        coo = tc4 > _f32(0.0)
        coo_f = jnp.where(coo, _f32(1.0), _f32(0.0))

        tx, ty, tw, th = tcol(0), tcol(1), tcol(2), tcol(3)
        t1x = tx / _f32(S) - _f32(0.5) * tw
        t2x = tx / _f32(S) + _f32(0.5) * tw
        t1y = ty / _f32(S) - _f32(0.5) * th
        t2y = ty / _f32(S) + _f32(0.5) * th
        a2 = (t2x - t1x) * (t2y - t1y)

        def iou(px, py, pw, ph):
            p1x = px / _f32(S) - _f32(0.5) * pw
            p2x = px / _f32(S) + _f32(0.5) * pw
            p1y = py / _f32(S) - _f32(0.5) * ph
            p2y = py / _f32(S) + _f32(0.5) * ph
            wx = jnp.maximum(
                jnp.minimum(p2x, t2x) - jnp.maximum(p1x, t1x), _f32(0.0))
            wy = jnp.maximum(
                jnp.minimum(p2y, t2y) - jnp.maximum(p1y, t1y), _f32(0.0))
            inter = wx * wy
            a1 = (p2x - p1x) * (p2y - p1y)
            denom = a1 + a2 - inter
            safe = jnp.where(coo, denom, _f32(1.0))
            return inter / safe

        iou0 = iou(pcol(0), pcol(1), pcol(2), pcol(3))
        iou1 = iou(pcol(5), pcol(6), pcol(7), pcol(8))
        max_iou = jnp.maximum(iou0, iou1)
        resp_c = jnp.where(iou1 > iou0, pc1, pc0)
        dc = resp_c - max_iou
        contain = dc * dc

        cls = jnp.zeros((L,), _f32)
        for c in range(C):
            d = pcol(10 + c) - tcol(10 + c)
            cls = cls + d * d

        accvm[...] = accvm[...] + coo_f * (contain + cls)


def _sc_body(pred_hbm, tgt_hbm, out_hbm, pvm, tvm, accvm, redvm, shared,
             sem_p, sem_t):
    sid = lax.axis_index("s")
    u0 = 12 * sid + jnp.minimum(sid, 4)
    cnt = jnp.where(sid < 4, 13, 12)
    slab0 = jnp.minimum(u0 // NG, NSLAB - NBUF)

    copies = []
    for j in range(NBUF):
        copies.append(pltpu.async_copy(
            pred_hbm.at[slab0 + j], pvm.at[pl.ds(j * LEN, LEN)], sem_p))
        copies.append(pltpu.async_copy(
            tgt_hbm.at[slab0 + j], tvm.at[pl.ds(j * LEN, LEN)], sem_t))
    for cp in copies:
        cp.wait()

    accvm[...] = jnp.zeros((L,), _f32)
    lane = lax.iota(jnp.int32, L)

    def unit_body(i, carry):
        u = u0 + i
        slab = u // NG
        jv = jnp.full((L,), (slab - slab0) * LEN, jnp.int32)
        bv = (u - slab * NG) * L + lane
        _unit_losses(pvm, tvm, jv, bv, accvm)
        return carry

    lax.fori_loop(0, cnt, unit_body, 0)

    pltpu.sync_copy(accvm, shared.at[pl.ds(sid * L, L)])
    plsc.subcore_barrier()

    @pl.when(sid == 0)
    def _():
        pltpu.sync_copy(shared, redvm)
        t = jnp.zeros((L,), _f32)
        for i in range(NS):
            t = t + redvm[pl.ds(i * L, L)]
        total = jnp.sum(t) * _f32(1.0 / BS)
        accvm[...] = jnp.full((L,), total, _f32)
        pltpu.sync_copy(accvm, out_hbm)


_mesh = plsc.VectorSubcoreMesh(
    core_axis_name="c", subcore_axis_name="s", num_cores=1)

_sc_yolo = functools.partial(
    pl.kernel,
    out_type=jax.ShapeDtypeStruct((L,), _f32),
    mesh=_mesh,
    compiler_params=pltpu.CompilerParams(
        needs_layout_passes=False, use_tc_tiling_on_sc=True),
    scratch_types=[
        pltpu.VMEM((NBUF * LEN, BS), _f32),
        pltpu.VMEM((NBUF * LEN, BS), _f32),
        pltpu.VMEM((L,), _f32),
        pltpu.VMEM((NS * L,), _f32),
        pltpu.VMEM_SHARED((NS * L,), _f32),
        pltpu.SemaphoreType.DMA,
        pltpu.SemaphoreType.DMA,
    ],
)(_sc_body)


def kernel(prediction, target):
    qp = jnp.transpose(prediction, (1, 2, 3, 0)).reshape(NSLAB, LEN, BS)
    qt = jnp.transpose(target, (1, 2, 3, 0)).reshape(NSLAB, LEN, BS)
    out = _sc_yolo(qp, qt)
    return out[0]


# speedup vs baseline: 1.2183x; 1.0032x over previous
import functools

import jax
import jax.numpy as jnp
from jax import lax
from jax.experimental import pallas as pl
from jax.experimental.pallas import tpu as pltpu
from jax.experimental.pallas import tpu_sc as plsc

S = 7
B = 2
C = 20
LEN = 5 * B + C
BS = 64
L = 16
NS = 16
NSLAB = S * S
NG = BS // L
NU = NSLAB * NG
NBUF = 5

_f32 = jnp.float32


_INV_S = 1.0 / S


def _unit_losses(pvm, tvm, jv, bv):
    def pcol(c):
        return plsc.load_gather(pvm, [jv + c, bv])

    def tcol(c):
        return plsc.load_gather(tvm, [jv + c, bv])

    tc4 = tcol(4)
    tc9 = tcol(9)
    pc0 = pcol(4)
    pc1 = pcol(9)

    noo_f = jnp.where(tc4 == _f32(0.0), _f32(1.0), _f32(0.0))
    d0 = pc0 - tc4
    d1 = pc1 - tc9
    noo = _f32(0.5) * noo_f * (d0 * d0 + d1 * d1)

    coo = tc4 > _f32(0.0)

    def coo_fn():
        coo_f = jnp.where(coo, _f32(1.0), _f32(0.0))

        tx, ty, tw, th = tcol(0), tcol(1), tcol(2), tcol(3)
        t1x = tx * _f32(_INV_S) - _f32(0.5) * tw
        t2x = tx * _f32(_INV_S) + _f32(0.5) * tw
        t1y = ty * _f32(_INV_S) - _f32(0.5) * th
        t2y = ty * _f32(_INV_S) + _f32(0.5) * th
        a2 = (t2x - t1x) * (t2y - t1y)

        def iou(px, py, pw, ph):
            p1x = px * _f32(_INV_S) - _f32(0.5) * pw
            p2x = px * _f32(_INV_S) + _f32(0.5) * pw
            p1y = py * _f32(_INV_S) - _f32(0.5) * ph
            p2y = py * _f32(_INV_S) + _f32(0.5) * ph
            wx = jnp.maximum(
                jnp.minimum(p2x, t2x) - jnp.maximum(p1x, t1x), _f32(0.0))
            wy = jnp.maximum(
                jnp.minimum(p2y, t2y) - jnp.maximum(p1y, t1y), _f32(0.0))
            inter = wx * wy
            a1 = (p2x - p1x) * (p2y - p1y)
            denom = a1 + a2 - inter
            safe = jnp.where(coo, denom, _f32(1.0))
            return inter / safe

        iou0 = iou(pcol(0), pcol(1), pcol(2), pcol(3))
        iou1 = iou(pcol(5), pcol(6), pcol(7), pcol(8))
        max_iou = jnp.maximum(iou0, iou1)
        resp_c = jnp.where(iou1 > iou0, pc1, pc0)
        dc = resp_c - max_iou
        contain = dc * dc

        cls = jnp.zeros((L,), _f32)
        for c in range(C):
            d = pcol(10 + c) - tcol(10 + c)
            cls = cls + d * d

        return coo_f * (contain + cls)

    cnt = plsc.all_reduce_population_count(coo)
    obj = lax.cond(cnt[0] > 0, coo_fn, lambda: jnp.zeros((L,), _f32))
    return noo + obj


def _sc_body(pred_hbm, tgt_hbm, out_hbm, pvm, tvm, accvm, redvm, shared,
             sem_p, sem_t):
    sid = lax.axis_index("s")
    u0 = 12 * sid + jnp.minimum(sid, 4)
    cnt = jnp.where(sid < 4, 13, 12)
    slab0 = jnp.minimum(u0 // NG, NSLAB - NBUF)

    copies = []
    for j in range(NBUF):
        copies.append(pltpu.async_copy(
            pred_hbm.at[slab0 + j], pvm.at[pl.ds(j * LEN, LEN)], sem_p))
        copies.append(pltpu.async_copy(
            tgt_hbm.at[slab0 + j], tvm.at[pl.ds(j * LEN, LEN)], sem_t))
    for cp in copies:
        cp.wait()

    lane = lax.iota(jnp.int32, L)

    def unit_body(i, acc):
        u = u0 + i
        slab = u // NG
        jv = jnp.full((L,), (slab - slab0) * LEN, jnp.int32)
        bv = (u - slab * NG) * L + lane
        return acc + _unit_losses(pvm, tvm, jv, bv)

    accvm[...] = lax.fori_loop(0, cnt, unit_body, jnp.zeros((L,), _f32))

    pltpu.sync_copy(accvm, shared.at[pl.ds(sid * L, L)])
    plsc.subcore_barrier()

    @pl.when(sid == 0)
    def _():
        pltpu.sync_copy(shared, redvm)
        t = jnp.zeros((L,), _f32)
        for i in range(NS):
            t = t + redvm[pl.ds(i * L, L)]
        total = jnp.sum(t) * _f32(1.0 / BS)
        accvm[...] = jnp.full((L,), total, _f32)
        pltpu.sync_copy(accvm, out_hbm)


_mesh = plsc.VectorSubcoreMesh(
    core_axis_name="c", subcore_axis_name="s", num_cores=1)

_sc_yolo = functools.partial(
    pl.kernel,
    out_type=jax.ShapeDtypeStruct((L,), _f32),
    mesh=_mesh,
    compiler_params=pltpu.CompilerParams(
        needs_layout_passes=False, use_tc_tiling_on_sc=True),
    scratch_types=[
        pltpu.VMEM((NBUF * LEN, BS), _f32),
        pltpu.VMEM((NBUF * LEN, BS), _f32),
        pltpu.VMEM((L,), _f32),
        pltpu.VMEM((NS * L,), _f32),
        pltpu.VMEM_SHARED((NS * L,), _f32),
        pltpu.SemaphoreType.DMA,
        pltpu.SemaphoreType.DMA,
    ],
)(_sc_body)


def kernel(prediction, target):
    qp = jnp.transpose(prediction, (1, 2, 3, 0)).reshape(NSLAB, LEN, BS)
    qt = jnp.transpose(target, (1, 2, 3, 0)).reshape(NSLAB, LEN, BS)
    out = _sc_yolo(qp, qt)
    return out[0]
